# baseline (speedup 1.0000x reference)
import dataclasses
import functools

import jax
import jax.numpy as jnp
from jax import lax
from jax.experimental import pallas as pl
from jax.experimental.pallas import tpu as pltpu
from jax.experimental.pallas import tpu_sc as plsc

N = 10000
E = 320000
D = 128

NC = 2
NS = 16
NW = NC * NS
LANES = 128
RPW = 80
SRC_ROWS = RPW + 8
E_ROWS = NW * RPW
RPW0 = 118
RPW1 = 40
F_ROWS = NS * (RPW0 + RPW1)
F_ROWS_AL = F_ROWS + 8
N_PAD = NS * 640
CROWS = N_PAD // D

_MESH = plsc.VectorSubcoreMesh(core_axis_name="c", subcore_axis_name="s",
                               num_cores=NC, num_subcores=NS)


def _sc_params():
    cp = pltpu.CompilerParams()
    if "needs_layout_passes" in pltpu.CompilerParams.__dataclass_fields__:
        cp = dataclasses.replace(cp, needs_layout_passes=False)
    return cp


def _segsum_body(h_hbm, src_hbm, dst_hbm, out_hbm,
                 srcv, dstv, srcv2, dstv2, rows, acc, sem, *sis):
    c = lax.axis_index("c")
    s = lax.axis_index("s")
    zv = jnp.zeros((16,), jnp.float32)

    @pl.loop(0, LANES)
    def _(i):
        @pl.loop(0, D, step=16)
        def _(j):
            rows[i, pl.ds(j, 16)] = zv

    @pl.loop(0, 5)
    def _(k):
        pltpu.sync_copy(rows, acc.at[pl.ds(s * 640 + k * LANES, LANES)])

    plsc.subcore_barrier()

    def work(base, nrows):
        def chunk(sv, dv):
            pltpu.async_copy(h_hbm.at[sv], rows, sem).wait()
            pltpu.sync_copy(rows, acc.at[dv], add=True)

        pltpu.async_copy(src_hbm.at[base], srcv, sis[0])
        pltpu.async_copy(dst_hbm.at[base], dstv, sis[1])

        @pl.loop(0, nrows, step=2)
        def _(j):
            pltpu.async_copy(src_hbm.at[base + j + 1], srcv2, sis[2])
            pltpu.async_copy(dst_hbm.at[base + j + 1], dstv2, sis[3])
            pltpu.make_async_copy(src_hbm.at[base], srcv, sis[0]).wait()
            pltpu.make_async_copy(dst_hbm.at[base], dstv, sis[1]).wait()
            chunk(srcv, dstv)
            pltpu.async_copy(src_hbm.at[base + j + 2], srcv, sis[0])
            pltpu.async_copy(dst_hbm.at[base + j + 2], dstv, sis[1])
            pltpu.make_async_copy(src_hbm.at[base], srcv2, sis[2]).wait()
            pltpu.make_async_copy(dst_hbm.at[base], dstv2, sis[3]).wait()
            chunk(srcv2, dstv2)

        pltpu.make_async_copy(src_hbm.at[base], srcv, sis[0]).wait()
        pltpu.make_async_copy(dst_hbm.at[base], dstv, sis[1]).wait()

    @pl.when(c == 0)
    def _():
        work(s * RPW0, RPW0)

    @pl.when(c == 1)
    def _():
        work(NS * RPW0 + s * RPW1, RPW1)

    plsc.subcore_barrier()

    pltpu.sync_copy(acc.at[pl.ds(s * 640, 640)],
                    out_hbm.at[c, pl.ds(s * 640, 640)])


_segsum = pl.kernel(
    _segsum_body,
    out_type=jax.ShapeDtypeStruct((NC, N_PAD, D), jnp.float32),
    mesh=_MESH,
    scratch_types=[
        pltpu.VMEM((LANES,), jnp.int32),
        pltpu.VMEM((LANES,), jnp.int32),
        pltpu.VMEM((LANES,), jnp.int32),
        pltpu.VMEM((LANES,), jnp.int32),
        pltpu.VMEM((LANES, D), jnp.float32),
        pltpu.VMEM_SHARED((N_PAD, D), jnp.float32),
        pltpu.SemaphoreType.DMA,
        pltpu.SemaphoreType.DMA,
        pltpu.SemaphoreType.DMA,
        pltpu.SemaphoreType.DMA,
        pltpu.SemaphoreType.DMA,
    ],
    compiler_params=_sc_params(),
    name="sc_segsum",
)


def _counts_body(dst_hbm, cnt_hbm, dsts, cntv, idxv, cacc, sem):
    c = lax.axis_index("c")
    s = lax.axis_index("s")
    wid = c * NS + s
    zv = jnp.zeros((16,), jnp.float32)
    ones16 = jnp.ones((16,), jnp.float32)

    pltpu.async_copy(dst_hbm.at[wid, pl.ds(0, RPW)], dsts, sem)

    @pl.loop(0, CROWS)
    def _(i):
        @pl.loop(0, D, step=16)
        def _(j):
            cntv[i, pl.ds(j, 16)] = zv

    @pl.when(s < 10)
    def _():
        pltpu.sync_copy(cntv.at[pl.ds(0, 8)], cacc.at[pl.ds(s * 8, 8)])

    @pl.loop(0, CROWS, step=16)
    def _(k):
        idxv[pl.ds(k, 16)] = lax.iota(jnp.int32, 16) + k

    pltpu.make_async_copy(dst_hbm.at[wid, pl.ds(0, RPW)], dsts, sem).wait()
    plsc.subcore_barrier()

    @pl.loop(0, RPW)
    def _(j):
        @pl.loop(0, LANES, step=16)
        def _(t):
            d16 = dsts[j, pl.ds(t, 16)]
            plsc.addupdate_scatter(
                cntv, [lax.shift_right_logical(d16, 7),
                       lax.bitwise_and(d16, 127)], ones16)

    pltpu.sync_copy(cntv, cacc.at[idxv], add=True)
    plsc.subcore_barrier()

    @pl.when(s < 10)
    def _():
        pltpu.sync_copy(cacc.at[pl.ds(s * 8, 8)],
                        cnt_hbm.at[c, pl.ds(s * 8, 8)])


_counts = pl.kernel(
    _counts_body,
    out_type=jax.ShapeDtypeStruct((NC, CROWS, D), jnp.float32),
    mesh=_MESH,
    scratch_types=[
        pltpu.VMEM((RPW, LANES), jnp.int32),
        pltpu.VMEM((CROWS, D), jnp.float32),
        pltpu.VMEM((CROWS,), jnp.int32),
        pltpu.VMEM_SHARED((CROWS, D), jnp.float32),
        pltpu.SemaphoreType.DMA,
    ],
    compiler_params=_sc_params(),
    name="sc_counts",
)

_BLK = 1024


def _root_body(h_ref, w_ref, o_ref):
    o_ref[...] = lax.dot_general(
        h_ref[...], w_ref[...], (((1,), (1,)), ((), ())),
        preferred_element_type=jnp.float32,
        precision=lax.Precision.HIGHEST)


def _root(h, w):
    return pl.pallas_call(
        _root_body,
        grid=(pl.cdiv(N, _BLK),),
        in_specs=[
            pl.BlockSpec((_BLK, D), lambda i: (i, 0)),
            pl.BlockSpec((D, D), lambda i: (0, 0)),
        ],
        out_specs=pl.BlockSpec((_BLK, D), lambda i: (i, 0)),
        out_shape=jax.ShapeDtypeStruct((N, D), jnp.float32),
    )(h, w)


def _combine_body(relu, p_ref, c_ref, hr_ref, w_ref, b_ref, o_ref):
    summed = p_ref[0] + p_ref[1]
    mean = summed / jnp.maximum(c_ref[...], 1.0)
    out = lax.dot_general(
        mean, w_ref[...], (((1,), (1,)), ((), ())),
        preferred_element_type=jnp.float32,
        precision=lax.Precision.HIGHEST)
    out = out + b_ref[...] + hr_ref[...]
    if relu:
        out = jnp.maximum(out, 0.0)
    o_ref[...] = out


def _combine(p, cnt, hr, wl, bl, relu):
    return pl.pallas_call(
        functools.partial(_combine_body, relu),
        grid=(pl.cdiv(N, _BLK),),
        in_specs=[
            pl.BlockSpec((NC, _BLK, D), lambda i: (0, i, 0)),
            pl.BlockSpec((_BLK, 1), lambda i: (i, 0)),
            pl.BlockSpec((_BLK, D), lambda i: (i, 0)),
            pl.BlockSpec((D, D), lambda i: (0, 0)),
            pl.BlockSpec((1, D), lambda i: (0, 0)),
        ],
        out_specs=pl.BlockSpec((_BLK, D), lambda i: (i, 0)),
        out_shape=jax.ShapeDtypeStruct((N, D), jnp.float32),
    )(p, cnt, hr, wl, bl.reshape(1, D))


def kernel(x, edge_index, W1l, b1l, W1r, W2l, b2l, W2r, W3l, b3l, W3r):
    pad = E_ROWS * LANES - E
    dst = jnp.concatenate([edge_index[1], jnp.full((pad,), N, jnp.int32)])
    dstp = jnp.pad(dst.reshape(NW, RPW, LANES),
                   ((0, 0), (0, SRC_ROWS - RPW), (0, 0)),
                   constant_values=N)

    padf = F_ROWS_AL * LANES - E
    srcf = jnp.concatenate([edge_index[0],
                            jnp.zeros((padf,), jnp.int32)]).reshape(F_ROWS_AL,
                                                                    LANES)
    dstf = jnp.concatenate([edge_index[1],
                            jnp.full((padf,), N, jnp.int32)]).reshape(F_ROWS_AL,
                                                                      LANES)

    cnt_grid = _counts(dstp)
    cnt = (cnt_grid[0] + cnt_grid[1]).reshape(N_PAD, 1)

    hr1 = _root(x, W1r)
    p1 = _segsum(x, srcf, dstf)
    h1 = _combine(p1, cnt, hr1, W1l, b1l, relu=True)

    hr2 = _root(h1, W2r)
    p2 = _segsum(h1, srcf, dstf)
    h2 = _combine(p2, cnt, hr2, W2l, b2l, relu=True)

    hr3 = _root(h2, W3r)
    p3 = _segsum(h2, srcf, dstf)
    return _combine(p3, cnt, hr3, W3l, b3l, relu=False)


# speedup vs baseline: 1.0147x; 1.0147x over previous
import dataclasses
import functools

import jax
import jax.numpy as jnp
from jax import lax
from jax.experimental import pallas as pl
from jax.experimental.pallas import tpu as pltpu
from jax.experimental.pallas import tpu_sc as plsc

N = 10000
E = 320000
D = 128

NC = 2
NS = 16
NW = NC * NS
LANES = 128
RPW = 80
SRC_ROWS = RPW + 8
E_ROWS = NW * RPW
RPW0 = 114
RPW1 = 44
F_ROWS = NS * (RPW0 + RPW1)
F_ROWS_AL = F_ROWS + 8
N_PAD = NS * 640
CROWS = N_PAD // D

_MESH = plsc.VectorSubcoreMesh(core_axis_name="c", subcore_axis_name="s",
                               num_cores=NC, num_subcores=NS)


def _sc_params():
    cp = pltpu.CompilerParams()
    if "needs_layout_passes" in pltpu.CompilerParams.__dataclass_fields__:
        cp = dataclasses.replace(cp, needs_layout_passes=False)
    return cp


def _segsum_body(h_hbm, src_hbm, dst_hbm, out_hbm,
                 srcv, dstv, srcv2, dstv2, rows, acc, sem, *sis):
    c = lax.axis_index("c")
    s = lax.axis_index("s")
    zv = jnp.zeros((16,), jnp.float32)

    @pl.loop(0, LANES)
    def _(i):
        @pl.loop(0, D, step=16)
        def _(j):
            rows[i, pl.ds(j, 16)] = zv

    @pl.loop(0, 5)
    def _(k):
        pltpu.sync_copy(rows, acc.at[pl.ds(s * 640 + k * LANES, LANES)])

    plsc.subcore_barrier()

    def work(base, nrows):
        def chunk(sv, dv):
            pltpu.async_copy(h_hbm.at[sv], rows, sem).wait()
            pltpu.sync_copy(rows, acc.at[dv], add=True)

        pltpu.async_copy(src_hbm.at[base], srcv, sis[0])
        pltpu.async_copy(dst_hbm.at[base], dstv, sis[1])

        @pl.loop(0, nrows, step=2)
        def _(j):
            pltpu.async_copy(src_hbm.at[base + j + 1], srcv2, sis[2])
            pltpu.async_copy(dst_hbm.at[base + j + 1], dstv2, sis[3])
            pltpu.make_async_copy(src_hbm.at[base], srcv, sis[0]).wait()
            pltpu.make_async_copy(dst_hbm.at[base], dstv, sis[1]).wait()
            chunk(srcv, dstv)
            pltpu.async_copy(src_hbm.at[base + j + 2], srcv, sis[0])
            pltpu.async_copy(dst_hbm.at[base + j + 2], dstv, sis[1])
            pltpu.make_async_copy(src_hbm.at[base], srcv2, sis[2]).wait()
            pltpu.make_async_copy(dst_hbm.at[base], dstv2, sis[3]).wait()
            chunk(srcv2, dstv2)

        pltpu.make_async_copy(src_hbm.at[base], srcv, sis[0]).wait()
        pltpu.make_async_copy(dst_hbm.at[base], dstv, sis[1]).wait()

    @pl.when(c == 0)
    def _():
        work(s * RPW0, RPW0)

    @pl.when(c == 1)
    def _():
        work(NS * RPW0 + s * RPW1, RPW1)

    plsc.subcore_barrier()

    pltpu.sync_copy(acc.at[pl.ds(s * 640, 640)],
                    out_hbm.at[c, pl.ds(s * 640, 640)])


_segsum = pl.kernel(
    _segsum_body,
    out_type=jax.ShapeDtypeStruct((NC, N_PAD, D), jnp.float32),
    mesh=_MESH,
    scratch_types=[
        pltpu.VMEM((LANES,), jnp.int32),
        pltpu.VMEM((LANES,), jnp.int32),
        pltpu.VMEM((LANES,), jnp.int32),
        pltpu.VMEM((LANES,), jnp.int32),
        pltpu.VMEM((LANES, D), jnp.float32),
        pltpu.VMEM_SHARED((N_PAD, D), jnp.float32),
        pltpu.SemaphoreType.DMA,
        pltpu.SemaphoreType.DMA,
        pltpu.SemaphoreType.DMA,
        pltpu.SemaphoreType.DMA,
        pltpu.SemaphoreType.DMA,
    ],
    compiler_params=_sc_params(),
    name="sc_segsum",
)


def _counts_body(dst_hbm, cnt_hbm, dsts, cntv, idxv, cacc, sem):
    c = lax.axis_index("c")
    s = lax.axis_index("s")
    wid = c * NS + s
    zv = jnp.zeros((16,), jnp.float32)
    ones16 = jnp.ones((16,), jnp.float32)

    pltpu.async_copy(dst_hbm.at[wid, pl.ds(0, RPW)], dsts, sem)

    @pl.loop(0, CROWS)
    def _(i):
        @pl.loop(0, D, step=16)
        def _(j):
            cntv[i, pl.ds(j, 16)] = zv

    @pl.when(s < 10)
    def _():
        pltpu.sync_copy(cntv.at[pl.ds(0, 8)], cacc.at[pl.ds(s * 8, 8)])

    @pl.loop(0, CROWS, step=16)
    def _(k):
        idxv[pl.ds(k, 16)] = lax.iota(jnp.int32, 16) + k

    pltpu.make_async_copy(dst_hbm.at[wid, pl.ds(0, RPW)], dsts, sem).wait()
    plsc.subcore_barrier()

    @pl.loop(0, RPW)
    def _(j):
        @pl.loop(0, LANES, step=16)
        def _(t):
            d16 = dsts[j, pl.ds(t, 16)]
            plsc.addupdate_scatter(
                cntv, [lax.shift_right_logical(d16, 7),
                       lax.bitwise_and(d16, 127)], ones16)

    pltpu.sync_copy(cntv, cacc.at[idxv], add=True)
    plsc.subcore_barrier()

    @pl.when(s < 10)
    def _():
        pltpu.sync_copy(cacc.at[pl.ds(s * 8, 8)],
                        cnt_hbm.at[c, pl.ds(s * 8, 8)])


_counts = pl.kernel(
    _counts_body,
    out_type=jax.ShapeDtypeStruct((NC, CROWS, D), jnp.float32),
    mesh=_MESH,
    scratch_types=[
        pltpu.VMEM((RPW, LANES), jnp.int32),
        pltpu.VMEM((CROWS, D), jnp.float32),
        pltpu.VMEM((CROWS,), jnp.int32),
        pltpu.VMEM_SHARED((CROWS, D), jnp.float32),
        pltpu.SemaphoreType.DMA,
    ],
    compiler_params=_sc_params(),
    name="sc_counts",
)

_BLK = 1024


def _root_body(h_ref, w_ref, o_ref):
    o_ref[...] = lax.dot_general(
        h_ref[...], w_ref[...], (((1,), (1,)), ((), ())),
        preferred_element_type=jnp.float32,
        precision=lax.Precision.HIGHEST)


def _root(h, w):
    return pl.pallas_call(
        _root_body,
        grid=(pl.cdiv(N, _BLK),),
        in_specs=[
            pl.BlockSpec((_BLK, D), lambda i: (i, 0)),
            pl.BlockSpec((D, D), lambda i: (0, 0)),
        ],
        out_specs=pl.BlockSpec((_BLK, D), lambda i: (i, 0)),
        out_shape=jax.ShapeDtypeStruct((N, D), jnp.float32),
    )(h, w)


def _combine_body(relu, p_ref, c_ref, hr_ref, w_ref, b_ref, o_ref):
    summed = p_ref[0] + p_ref[1]
    mean = summed / jnp.maximum(c_ref[...], 1.0)
    out = lax.dot_general(
        mean, w_ref[...], (((1,), (1,)), ((), ())),
        preferred_element_type=jnp.float32,
        precision=lax.Precision.HIGHEST)
    out = out + b_ref[...] + hr_ref[...]
    if relu:
        out = jnp.maximum(out, 0.0)
    o_ref[...] = out


def _combine(p, cnt, hr, wl, bl, relu):
    return pl.pallas_call(
        functools.partial(_combine_body, relu),
        grid=(pl.cdiv(N, _BLK),),
        in_specs=[
            pl.BlockSpec((NC, _BLK, D), lambda i: (0, i, 0)),
            pl.BlockSpec((_BLK, 1), lambda i: (i, 0)),
            pl.BlockSpec((_BLK, D), lambda i: (i, 0)),
            pl.BlockSpec((D, D), lambda i: (0, 0)),
            pl.BlockSpec((1, D), lambda i: (0, 0)),
        ],
        out_specs=pl.BlockSpec((_BLK, D), lambda i: (i, 0)),
        out_shape=jax.ShapeDtypeStruct((N, D), jnp.float32),
    )(p, cnt, hr, wl, bl.reshape(1, D))


def kernel(x, edge_index, W1l, b1l, W1r, W2l, b2l, W2r, W3l, b3l, W3r):
    pad = E_ROWS * LANES - E
    dst = jnp.concatenate([edge_index[1], jnp.full((pad,), N, jnp.int32)])
    dstp = jnp.pad(dst.reshape(NW, RPW, LANES),
                   ((0, 0), (0, SRC_ROWS - RPW), (0, 0)),
                   constant_values=N)

    padf = F_ROWS_AL * LANES - E
    srcf = jnp.concatenate([edge_index[0],
                            jnp.zeros((padf,), jnp.int32)]).reshape(F_ROWS_AL,
                                                                    LANES)
    dstf = jnp.concatenate([edge_index[1],
                            jnp.full((padf,), N, jnp.int32)]).reshape(F_ROWS_AL,
                                                                      LANES)

    cnt_grid = _counts(dstp)
    cnt = (cnt_grid[0] + cnt_grid[1]).reshape(N_PAD, 1)

    hr1 = _root(x, W1r)
    p1 = _segsum(x, srcf, dstf)
    h1 = _combine(p1, cnt, hr1, W1l, b1l, relu=True)

    hr2 = _root(h1, W2r)
    p2 = _segsum(h1, srcf, dstf)
    h2 = _combine(p2, cnt, hr2, W2l, b2l, relu=True)

    hr3 = _root(h2, W3r)
    p3 = _segsum(h2, srcf, dstf)
    return _combine(p3, cnt, hr3, W3l, b3l, relu=False)


# speedup vs baseline: 1.0245x; 1.0097x over previous
import dataclasses
import functools

import jax
import jax.numpy as jnp
from jax import lax
from jax.experimental import pallas as pl
from jax.experimental.pallas import tpu as pltpu
from jax.experimental.pallas import tpu_sc as plsc

N = 10000
E = 320000
D = 128

NC = 2
NS = 16
NW = NC * NS
LANES = 128
RPW = 80
SRC_ROWS = RPW + 8
E_ROWS = NW * RPW
RPW0 = 110
RPW1 = 48
F_ROWS = NS * (RPW0 + RPW1)
F_ROWS_AL = F_ROWS + 8
N_PAD = NS * 640
CROWS = N_PAD // D

_MESH = plsc.VectorSubcoreMesh(core_axis_name="c", subcore_axis_name="s",
                               num_cores=NC, num_subcores=NS)


def _sc_params():
    cp = pltpu.CompilerParams()
    if "needs_layout_passes" in pltpu.CompilerParams.__dataclass_fields__:
        cp = dataclasses.replace(cp, needs_layout_passes=False)
    return cp


def _segsum_body(h_hbm, src_hbm, dst_hbm, out_hbm,
                 srcv, dstv, srcv2, dstv2, rows, acc, sem, *sis):
    c = lax.axis_index("c")
    s = lax.axis_index("s")
    zv = jnp.zeros((16,), jnp.float32)

    @pl.loop(0, LANES)
    def _(i):
        @pl.loop(0, D, step=16)
        def _(j):
            rows[i, pl.ds(j, 16)] = zv

    @pl.loop(0, 5)
    def _(k):
        pltpu.sync_copy(rows, acc.at[pl.ds(s * 640 + k * LANES, LANES)])

    plsc.subcore_barrier()

    def work(base, nrows):
        def chunk(sv, dv):
            pltpu.async_copy(h_hbm.at[sv], rows, sem).wait()
            pltpu.sync_copy(rows, acc.at[dv], add=True)

        pltpu.async_copy(src_hbm.at[base], srcv, sis[0])
        pltpu.async_copy(dst_hbm.at[base], dstv, sis[1])

        @pl.loop(0, nrows, step=2)
        def _(j):
            pltpu.async_copy(src_hbm.at[base + j + 1], srcv2, sis[2])
            pltpu.async_copy(dst_hbm.at[base + j + 1], dstv2, sis[3])
            pltpu.make_async_copy(src_hbm.at[base], srcv, sis[0]).wait()
            pltpu.make_async_copy(dst_hbm.at[base], dstv, sis[1]).wait()
            chunk(srcv, dstv)
            pltpu.async_copy(src_hbm.at[base + j + 2], srcv, sis[0])
            pltpu.async_copy(dst_hbm.at[base + j + 2], dstv, sis[1])
            pltpu.make_async_copy(src_hbm.at[base], srcv2, sis[2]).wait()
            pltpu.make_async_copy(dst_hbm.at[base], dstv2, sis[3]).wait()
            chunk(srcv2, dstv2)

        pltpu.make_async_copy(src_hbm.at[base], srcv, sis[0]).wait()
        pltpu.make_async_copy(dst_hbm.at[base], dstv, sis[1]).wait()

    @pl.when(c == 0)
    def _():
        work(s * RPW0, RPW0)

    @pl.when(c == 1)
    def _():
        work(NS * RPW0 + s * RPW1, RPW1)

    plsc.subcore_barrier()

    pltpu.sync_copy(acc.at[pl.ds(s * 640, 640)],
                    out_hbm.at[c, pl.ds(s * 640, 640)])


_segsum = pl.kernel(
    _segsum_body,
    out_type=jax.ShapeDtypeStruct((NC, N_PAD, D), jnp.float32),
    mesh=_MESH,
    scratch_types=[
        pltpu.VMEM((LANES,), jnp.int32),
        pltpu.VMEM((LANES,), jnp.int32),
        pltpu.VMEM((LANES,), jnp.int32),
        pltpu.VMEM((LANES,), jnp.int32),
        pltpu.VMEM((LANES, D), jnp.float32),
        pltpu.VMEM_SHARED((N_PAD, D), jnp.float32),
        pltpu.SemaphoreType.DMA,
        pltpu.SemaphoreType.DMA,
        pltpu.SemaphoreType.DMA,
        pltpu.SemaphoreType.DMA,
        pltpu.SemaphoreType.DMA,
    ],
    compiler_params=_sc_params(),
    name="sc_segsum",
)


def _counts_body(dst_hbm, cnt_hbm, dsts, cntv, idxv, cacc, sem):
    c = lax.axis_index("c")
    s = lax.axis_index("s")
    wid = c * NS + s
    zv = jnp.zeros((16,), jnp.float32)
    ones16 = jnp.ones((16,), jnp.float32)

    pltpu.async_copy(dst_hbm.at[wid, pl.ds(0, RPW)], dsts, sem)

    @pl.loop(0, CROWS)
    def _(i):
        @pl.loop(0, D, step=16)
        def _(j):
            cntv[i, pl.ds(j, 16)] = zv

    @pl.when(s < 10)
    def _():
        pltpu.sync_copy(cntv.at[pl.ds(0, 8)], cacc.at[pl.ds(s * 8, 8)])

    @pl.loop(0, CROWS, step=16)
    def _(k):
        idxv[pl.ds(k, 16)] = lax.iota(jnp.int32, 16) + k

    pltpu.make_async_copy(dst_hbm.at[wid, pl.ds(0, RPW)], dsts, sem).wait()
    plsc.subcore_barrier()

    @pl.loop(0, RPW)
    def _(j):
        @pl.loop(0, LANES, step=16)
        def _(t):
            d16 = dsts[j, pl.ds(t, 16)]
            plsc.addupdate_scatter(
                cntv, [lax.shift_right_logical(d16, 7),
                       lax.bitwise_and(d16, 127)], ones16)

    pltpu.sync_copy(cntv, cacc.at[idxv], add=True)
    plsc.subcore_barrier()

    @pl.when(s < 10)
    def _():
        pltpu.sync_copy(cacc.at[pl.ds(s * 8, 8)],
                        cnt_hbm.at[c, pl.ds(s * 8, 8)])


_counts = pl.kernel(
    _counts_body,
    out_type=jax.ShapeDtypeStruct((NC, CROWS, D), jnp.float32),
    mesh=_MESH,
    scratch_types=[
        pltpu.VMEM((RPW, LANES), jnp.int32),
        pltpu.VMEM((CROWS, D), jnp.float32),
        pltpu.VMEM((CROWS,), jnp.int32),
        pltpu.VMEM_SHARED((CROWS, D), jnp.float32),
        pltpu.SemaphoreType.DMA,
    ],
    compiler_params=_sc_params(),
    name="sc_counts",
)

_BLK = 1024


def _root_body(h_ref, w_ref, o_ref):
    o_ref[...] = lax.dot_general(
        h_ref[...], w_ref[...], (((1,), (1,)), ((), ())),
        preferred_element_type=jnp.float32,
        precision=lax.Precision.HIGHEST)


def _root(h, w):
    return pl.pallas_call(
        _root_body,
        grid=(pl.cdiv(N, _BLK),),
        in_specs=[
            pl.BlockSpec((_BLK, D), lambda i: (i, 0)),
            pl.BlockSpec((D, D), lambda i: (0, 0)),
        ],
        out_specs=pl.BlockSpec((_BLK, D), lambda i: (i, 0)),
        out_shape=jax.ShapeDtypeStruct((N, D), jnp.float32),
    )(h, w)


def _combine_body(relu, p_ref, c_ref, hr_ref, w_ref, b_ref, o_ref):
    summed = p_ref[0] + p_ref[1]
    mean = summed / jnp.maximum(c_ref[...], 1.0)
    out = lax.dot_general(
        mean, w_ref[...], (((1,), (1,)), ((), ())),
        preferred_element_type=jnp.float32,
        precision=lax.Precision.HIGHEST)
    out = out + b_ref[...] + hr_ref[...]
    if relu:
        out = jnp.maximum(out, 0.0)
    o_ref[...] = out


def _combine(p, cnt, hr, wl, bl, relu):
    return pl.pallas_call(
        functools.partial(_combine_body, relu),
        grid=(pl.cdiv(N, _BLK),),
        in_specs=[
            pl.BlockSpec((NC, _BLK, D), lambda i: (0, i, 0)),
            pl.BlockSpec((_BLK, 1), lambda i: (i, 0)),
            pl.BlockSpec((_BLK, D), lambda i: (i, 0)),
            pl.BlockSpec((D, D), lambda i: (0, 0)),
            pl.BlockSpec((1, D), lambda i: (0, 0)),
        ],
        out_specs=pl.BlockSpec((_BLK, D), lambda i: (i, 0)),
        out_shape=jax.ShapeDtypeStruct((N, D), jnp.float32),
    )(p, cnt, hr, wl, bl.reshape(1, D))


def kernel(x, edge_index, W1l, b1l, W1r, W2l, b2l, W2r, W3l, b3l, W3r):
    pad = E_ROWS * LANES - E
    dst = jnp.concatenate([edge_index[1], jnp.full((pad,), N, jnp.int32)])
    dstp = jnp.pad(dst.reshape(NW, RPW, LANES),
                   ((0, 0), (0, SRC_ROWS - RPW), (0, 0)),
                   constant_values=N)

    padf = F_ROWS_AL * LANES - E
    srcf = jnp.concatenate([edge_index[0],
                            jnp.zeros((padf,), jnp.int32)]).reshape(F_ROWS_AL,
                                                                    LANES)
    dstf = jnp.concatenate([edge_index[1],
                            jnp.full((padf,), N, jnp.int32)]).reshape(F_ROWS_AL,
                                                                      LANES)

    cnt_grid = _counts(dstp)
    cnt = (cnt_grid[0] + cnt_grid[1]).reshape(N_PAD, 1)

    hr1 = _root(x, W1r)
    p1 = _segsum(x, srcf, dstf)
    h1 = _combine(p1, cnt, hr1, W1l, b1l, relu=True)

    hr2 = _root(h1, W2r)
    p2 = _segsum(h1, srcf, dstf)
    h2 = _combine(p2, cnt, hr2, W2l, b2l, relu=True)

    hr3 = _root(h2, W3r)
    p3 = _segsum(h2, srcf, dstf)
    return _combine(p3, cnt, hr3, W3l, b3l, relu=False)
